# baseline (speedup 1.0000x reference)
import functools

import jax
import jax.numpy as jnp
from jax import lax
from jax.experimental import pallas as pl
from jax.experimental.pallas import tpu as pltpu
from jax.experimental.pallas import tpu_sc as plsc

N_NODES = 10000
D = 128
N_EDGES = 320000

NC = 2
NS = 16
NW = NC * NS

CHUNK = 128
CH = -(-N_EDGES // (NW * CHUNK))
CH0 = 118
CH1 = 2 * CH - CH0
E_PAD = NW * CH * CHUNK
ROWS_PER_TILE = 640
N_PAD = NS * ROWS_PER_TILE
DUMMY_ROW = N_NODES

_MESH = plsc.VectorSubcoreMesh(core_axis_name="c", subcore_axis_name="s")


@functools.partial(
    pl.kernel,
    mesh=_MESH,
    out_type=jax.ShapeDtypeStruct((NC, N_PAD, D), jnp.float32),
    scratch_types=[
        pltpu.VMEM((CH0, CHUNK), jnp.int32),
        pltpu.VMEM((CH0, CHUNK), jnp.int32),
        pltpu.VMEM((CHUNK, D), jnp.float32),
        pltpu.VMEM_SHARED((N_PAD, D), jnp.float32),
        pltpu.SemaphoreType.DMA,
    ],
)
def _sc_segment_sum(x_hbm, srcA_hbm, dstA_hbm, srcB_hbm, dstB_hbm, out_hbm,
                    src_v, dst_v, rows_v, acc, sem):
    c = lax.axis_index("c")
    s = lax.axis_index("s")

    @pl.when(c == 0)
    def _stage_a():
        pltpu.sync_copy(srcA_hbm.at[s], src_v.at[pl.ds(0, CH0)])
        pltpu.sync_copy(dstA_hbm.at[s], dst_v.at[pl.ds(0, CH0)])

    @pl.when(c == 1)
    def _stage_b():
        pltpu.sync_copy(srcB_hbm.at[s], src_v.at[pl.ds(0, CH1)])
        pltpu.sync_copy(dstB_hbm.at[s], dst_v.at[pl.ds(0, CH1)])

    n_ch = jnp.where(c == 0, CH0, CH1)

    zvec = jnp.zeros((16,), jnp.float32)

    def _zero_row(r, carry):
        for cc in range(D // 16):
            rows_v[r, pl.ds(cc * 16, 16)] = zvec
        return carry

    lax.fori_loop(0, CHUNK, _zero_row, 0)
    row0 = s * ROWS_PER_TILE

    def _zero_body(k, carry):
        pltpu.sync_copy(rows_v, acc.at[pl.ds(row0 + k * CHUNK, CHUNK)])
        return carry

    lax.fori_loop(0, ROWS_PER_TILE // CHUNK, _zero_body, 0)
    plsc.subcore_barrier()

    def _edge_body(j, carry):
        pltpu.async_copy(x_hbm.at[src_v.at[j]], rows_v, sem).wait()
        pltpu.sync_copy(rows_v, acc.at[dst_v.at[j]], add=True)
        return carry

    lax.fori_loop(0, n_ch, _edge_body, 0)
    plsc.subcore_barrier()

    pltpu.sync_copy(acc.at[pl.ds(row0, ROWS_PER_TILE)],
                    out_hbm.at[c].at[pl.ds(row0, ROWS_PER_TILE)])


_RB = 2000


def _dense_body(relu, p_ref, x_ref, wrT_ref, wrootT_ref, b_ref, o_ref):
    agg = p_ref[0] + p_ref[1]
    y = jnp.dot(agg, wrT_ref[...], preferred_element_type=jnp.float32)
    y = y + jnp.dot(x_ref[...], wrootT_ref[...],
                    preferred_element_type=jnp.float32)
    y = y + b_ref[...]
    if relu:
        y = jnp.maximum(y, 0.0)
    o_ref[...] = y


def _dense(p, x, wrT, wrootT, b, relu):
    return pl.pallas_call(
        functools.partial(_dense_body, relu),
        grid=(N_NODES // _RB,),
        in_specs=[
            pl.BlockSpec((NC, _RB, D), lambda i: (0, i, 0)),
            pl.BlockSpec((_RB, D), lambda i: (i, 0)),
            pl.BlockSpec((D, D), lambda i: (0, 0)),
            pl.BlockSpec((D, D), lambda i: (0, 0)),
            pl.BlockSpec((1, D), lambda i: (0, 0)),
        ],
        out_specs=pl.BlockSpec((_RB, D), lambda i: (i, 0)),
        out_shape=jax.ShapeDtypeStruct((N_NODES, D), jnp.float32),
    )(p, x, wrT, wrootT, b)


def kernel(x, edge_index, W1_rel, b1_rel, W1_root, W2_rel, b2_rel, W2_root):
    ei = edge_index.astype(jnp.int32)
    pad = E_PAD - N_EDGES
    src = jnp.concatenate([ei[0], jnp.zeros((pad,), jnp.int32)])
    dst = jnp.concatenate([ei[1], jnp.full((pad,), DUMMY_ROW, jnp.int32)])
    cut = NS * CH0 * CHUNK
    srcA = src[:cut].reshape(NS, CH0, CHUNK)
    dstA = dst[:cut].reshape(NS, CH0, CHUNK)
    srcB = src[cut:].reshape(NS, CH1, CHUNK)
    dstB = dst[cut:].reshape(NS, CH1, CHUNK)

    p1 = _sc_segment_sum(x, srcA, dstA, srcB, dstB)
    h = _dense(p1, x, W1_rel.T, W1_root.T, b1_rel.reshape(1, D), True)
    p2 = _sc_segment_sum(h, srcA, dstA, srcB, dstB)
    return _dense(p2, h, W2_rel.T, W2_root.T, b2_rel.reshape(1, D), False)


# speedup vs baseline: 1.0223x; 1.0223x over previous
import functools

import jax
import jax.numpy as jnp
from jax import lax
from jax.experimental import pallas as pl
from jax.experimental.pallas import tpu as pltpu
from jax.experimental.pallas import tpu_sc as plsc

N_NODES = 10000
D = 128
N_EDGES = 320000

NC = 2
NS = 16
NW = NC * NS

CHUNK = 128
CH = -(-N_EDGES // (NW * CHUNK))
CH0 = 110
CH1 = 2 * CH - CH0
E_PAD = NW * CH * CHUNK
ROWS_PER_TILE = 640
N_PAD = NS * ROWS_PER_TILE
DUMMY_ROW = N_NODES

_MESH = plsc.VectorSubcoreMesh(core_axis_name="c", subcore_axis_name="s")


@functools.partial(
    pl.kernel,
    mesh=_MESH,
    out_type=jax.ShapeDtypeStruct((NC, N_PAD, D), jnp.float32),
    scratch_types=[
        pltpu.VMEM((CH0, CHUNK), jnp.int32),
        pltpu.VMEM((CH0, CHUNK), jnp.int32),
        pltpu.VMEM((CHUNK, D), jnp.float32),
        pltpu.VMEM_SHARED((N_PAD, D), jnp.float32),
        pltpu.SemaphoreType.DMA,
    ],
)
def _sc_segment_sum(x_hbm, srcA_hbm, dstA_hbm, srcB_hbm, dstB_hbm, out_hbm,
                    src_v, dst_v, rows_v, acc, sem):
    c = lax.axis_index("c")
    s = lax.axis_index("s")

    @pl.when(c == 0)
    def _stage_a():
        pltpu.sync_copy(srcA_hbm.at[s], src_v.at[pl.ds(0, CH0)])
        pltpu.sync_copy(dstA_hbm.at[s], dst_v.at[pl.ds(0, CH0)])

    @pl.when(c == 1)
    def _stage_b():
        pltpu.sync_copy(srcB_hbm.at[s], src_v.at[pl.ds(0, CH1)])
        pltpu.sync_copy(dstB_hbm.at[s], dst_v.at[pl.ds(0, CH1)])

    n_ch = jnp.where(c == 0, CH0, CH1)

    zvec = jnp.zeros((16,), jnp.float32)

    def _zero_row(r, carry):
        for cc in range(D // 16):
            rows_v[r, pl.ds(cc * 16, 16)] = zvec
        return carry

    lax.fori_loop(0, CHUNK, _zero_row, 0)
    row0 = s * ROWS_PER_TILE

    def _zero_body(k, carry):
        pltpu.sync_copy(rows_v, acc.at[pl.ds(row0 + k * CHUNK, CHUNK)])
        return carry

    lax.fori_loop(0, ROWS_PER_TILE // CHUNK, _zero_body, 0)
    plsc.subcore_barrier()

    def _edge_body(j, carry):
        pltpu.async_copy(x_hbm.at[src_v.at[j]], rows_v, sem).wait()
        pltpu.sync_copy(rows_v, acc.at[dst_v.at[j]], add=True)
        return carry

    lax.fori_loop(0, n_ch, _edge_body, 0)
    plsc.subcore_barrier()

    pltpu.sync_copy(acc.at[pl.ds(row0, ROWS_PER_TILE)],
                    out_hbm.at[c].at[pl.ds(row0, ROWS_PER_TILE)])


_RB = 2000


def _dense_body(relu, p_ref, x_ref, wrT_ref, wrootT_ref, b_ref, o_ref):
    agg = p_ref[0] + p_ref[1]
    y = jnp.dot(agg, wrT_ref[...], preferred_element_type=jnp.float32)
    y = y + jnp.dot(x_ref[...], wrootT_ref[...],
                    preferred_element_type=jnp.float32)
    y = y + b_ref[...]
    if relu:
        y = jnp.maximum(y, 0.0)
    o_ref[...] = y


def _dense(p, x, wrT, wrootT, b, relu):
    return pl.pallas_call(
        functools.partial(_dense_body, relu),
        grid=(N_NODES // _RB,),
        in_specs=[
            pl.BlockSpec((NC, _RB, D), lambda i: (0, i, 0)),
            pl.BlockSpec((_RB, D), lambda i: (i, 0)),
            pl.BlockSpec((D, D), lambda i: (0, 0)),
            pl.BlockSpec((D, D), lambda i: (0, 0)),
            pl.BlockSpec((1, D), lambda i: (0, 0)),
        ],
        out_specs=pl.BlockSpec((_RB, D), lambda i: (i, 0)),
        out_shape=jax.ShapeDtypeStruct((N_NODES, D), jnp.float32),
    )(p, x, wrT, wrootT, b)


def kernel(x, edge_index, W1_rel, b1_rel, W1_root, W2_rel, b2_rel, W2_root):
    ei = edge_index.astype(jnp.int32)
    pad = E_PAD - N_EDGES
    src = jnp.concatenate([ei[0], jnp.zeros((pad,), jnp.int32)])
    dst = jnp.concatenate([ei[1], jnp.full((pad,), DUMMY_ROW, jnp.int32)])
    cut = NS * CH0 * CHUNK
    srcA = src[:cut].reshape(NS, CH0, CHUNK)
    dstA = dst[:cut].reshape(NS, CH0, CHUNK)
    srcB = src[cut:].reshape(NS, CH1, CHUNK)
    dstB = dst[cut:].reshape(NS, CH1, CHUNK)

    p1 = _sc_segment_sum(x, srcA, dstA, srcB, dstB)
    h = _dense(p1, x, W1_rel.T, W1_root.T, b1_rel.reshape(1, D), True)
    p2 = _sc_segment_sum(h, srcA, dstA, srcB, dstB)
    return _dense(p2, h, W2_rel.T, W2_root.T, b2_rel.reshape(1, D), False)
